# baseline (speedup 1.0000x reference)
import jax
import jax.numpy as jnp
from jax import lax
from jax.experimental import pallas as pl
from jax.experimental.pallas import tpu as pltpu
from jax.experimental.pallas import tpu_sc as plsc

NUM_CLASSES = 1000
HIDDEN = 1152
BATCH = 16384

_INFO = plsc.get_sparse_core_info()
NC = _INFO.num_cores
NS = _INFO.num_subcores
NW = NC * NS
B_PER_W = BATCH // NW
CHUNK = 32
NCHUNK = B_PER_W // CHUNK
NBUF = 3


def _embed_body(table_hbm, labels_hbm, out_hbm, idx_v, rows_a, rows_b, rows_c,
                gsem_a, gsem_b, gsem_c, wsem_a, wsem_b, wsem_c):
    wid = lax.axis_index("s") * NC + lax.axis_index("c")
    base = wid * B_PER_W

    pltpu.sync_copy(labels_hbm.at[pl.ds(base, B_PER_W)], idx_v)

    bufs = (rows_a, rows_b, rows_c)
    gsems = (gsem_a, gsem_b, gsem_c)
    wsems = (wsem_a, wsem_b, wsem_c)
    gcp = [None] * NBUF
    wcp = [None] * NBUF

    def gather(j):
        return pltpu.async_copy(
            table_hbm.at[idx_v.at[pl.ds(j * CHUNK, CHUNK)]],
            bufs[j % NBUF], gsems[j % NBUF])

    gcp[0] = gather(0)
    for i in range(NCHUNK):
        b = i % NBUF
        j = i + 1
        if j < NCHUNK:
            nb = j % NBUF
            if wcp[nb] is not None:
                wcp[nb].wait()
            gcp[nb] = gather(j)
        gcp[b].wait()
        wcp[b] = pltpu.async_copy(
            bufs[b], out_hbm.at[pl.ds(base + i * CHUNK, CHUNK)], wsems[b])
    for b in range(NBUF):
        if wcp[b] is not None:
            wcp[b].wait()


@jax.jit
def _embed(labels, embedding_table):
    mesh = plsc.VectorSubcoreMesh(core_axis_name="c", subcore_axis_name="s")
    f = pl.kernel(
        _embed_body,
        out_type=jax.ShapeDtypeStruct((BATCH, HIDDEN), jnp.float32),
        mesh=mesh,
        scratch_types=[
            pltpu.VMEM((B_PER_W,), jnp.int32),
            pltpu.VMEM((CHUNK, HIDDEN), jnp.float32),
            pltpu.VMEM((CHUNK, HIDDEN), jnp.float32),
            pltpu.VMEM((CHUNK, HIDDEN), jnp.float32),
            pltpu.SemaphoreType.DMA,
            pltpu.SemaphoreType.DMA,
            pltpu.SemaphoreType.DMA,
            pltpu.SemaphoreType.DMA,
            pltpu.SemaphoreType.DMA,
            pltpu.SemaphoreType.DMA,
        ],
    )
    return f(embedding_table, labels)


def kernel(labels, embedding_table):
    return _embed(labels.astype(jnp.int32), embedding_table)
